# baseline (speedup 1.0000x reference)
import functools

import jax
import jax.numpy as jnp
from jax import lax
from jax.experimental import pallas as pl
from jax.experimental.pallas import tpu as pltpu
from jax.experimental.pallas import tpu_sc as plsc

_NC = 2
_NS = 16
_NW = _NC * _NS
_BATCH = 16384
_FEAT = 64
_BPW = _BATCH // _NW
_IDX_CHUNK = 128
_NCHUNK = _BPW // _IDX_CHUNK
_NGROUP = _BPW // 16


def _vec_sqrt(s):
    s = jnp.maximum(s, jnp.float32(1e-30))
    i = plsc.bitcast(s, jnp.int32)
    i = jnp.int32(0x5F3759DF) - (i >> 1)
    y = plsc.bitcast(i, jnp.float32)
    for _ in range(3):
        y = y * (jnp.float32(1.5) - jnp.float32(0.5) * s * y * y)
    return s * y


def _body(labels2d, x_hbm, centre_hbm, out_hbm, idx0, idx1, idx2, idx3, x_v, c_v, res_v, sem):
    wid = lax.axis_index("s") * _NC + lax.axis_index("c")
    base = wid * _BPW
    idx_refs = (idx0, idx1, idx2, idx3)

    for j in range(_NCHUNK):
        pltpu.sync_copy(labels2d.at[wid * _NCHUNK + j], idx_refs[j])

    copies = [
        pltpu.async_copy(
            centre_hbm.at[idx_refs[j]],
            c_v.at[pl.ds(j * _IDX_CHUNK, _IDX_CHUNK), :],
            sem,
        )
        for j in range(_NCHUNK)
    ]
    pltpu.sync_copy(x_hbm.at[pl.ds(base, _BPW), :], x_v)
    for cp in copies:
        cp.wait()

    def group(g, tot):
        rv = g * 16 + lax.iota(jnp.int32, 16)
        acc = jnp.zeros((16,), jnp.float32)
        for f in range(_FEAT):
            cvec = jnp.full((16,), f, jnp.int32)
            gx = plsc.load_gather(x_v, [rv, cvec])
            gc = plsc.load_gather(c_v, [rv, cvec])
            d = gx - gc
            acc = acc + d * d
        return tot + _vec_sqrt(acc)

    total = lax.fori_loop(0, _NGROUP, group, jnp.zeros((16,), jnp.float32))
    res_v[...] = jnp.full((16,), jnp.sum(total), jnp.float32)
    pltpu.sync_copy(res_v, out_hbm.at[wid])


_sc_call = functools.partial(
    pl.kernel,
    out_type=jax.ShapeDtypeStruct((_NW, 16), jnp.float32),
    mesh=plsc.VectorSubcoreMesh(
        core_axis_name="c", subcore_axis_name="s", num_cores=_NC, num_subcores=_NS
    ),
    compiler_params=pltpu.CompilerParams(
        needs_layout_passes=False, use_tc_tiling_on_sc=False
    ),
    scratch_types=[
        pltpu.VMEM((_IDX_CHUNK,), jnp.int32),
        pltpu.VMEM((_IDX_CHUNK,), jnp.int32),
        pltpu.VMEM((_IDX_CHUNK,), jnp.int32),
        pltpu.VMEM((_IDX_CHUNK,), jnp.int32),
        pltpu.VMEM((_BPW, _FEAT), jnp.float32),
        pltpu.VMEM((_BPW, _FEAT), jnp.float32),
        pltpu.VMEM((16,), jnp.float32),
        pltpu.SemaphoreType.DMA,
    ],
)


def kernel(x, labels, centre):
    labels2d = labels.astype(jnp.int32).reshape(_BATCH // _IDX_CHUNK, _IDX_CHUNK)
    partials = _sc_call(_body)(labels2d, x, centre)
    return jnp.sum(partials[:, 0])


# speedup vs baseline: 1.6156x; 1.6156x over previous
import functools

import jax
import jax.numpy as jnp
from jax import lax
from jax.experimental import pallas as pl
from jax.experimental.pallas import tpu as pltpu
from jax.experimental.pallas import tpu_sc as plsc

_NC = 2
_NS = 16
_NW = _NC * _NS
_FEAT = 64
_BATCH = 16384
_CLS = 100000
_BLK = 2048
_NBLK = _BATCH // _BLK
_U = 8


def _sc_body(ct, xt, labels, d2_out, row_v, lab_v, x_v, acc_v, sem):
    c = lax.axis_index("c")
    s = lax.axis_index("s")
    w = c * _NS + s

    for fi in range(2):
        f = w * 2 + fi
        pltpu.sync_copy(ct.at[f], row_v)
        for b in range(_NBLK):
            pltpu.sync_copy(labels.at[pl.ds(b * _BLK, _BLK)], lab_v)
            pltpu.sync_copy(xt.at[f, pl.ds(b * _BLK, _BLK)], x_v)

            def grp(i, _, b=b, fi=fi):
                base = i * (16 * _U)
                for u in range(_U):
                    o = base + u * 16
                    lv = lab_v[pl.ds(o, 16)]
                    xv = x_v[pl.ds(o, 16)]
                    cv = plsc.load_gather(row_v, [lv])
                    d = xv - cv
                    ao = b * _BLK + o
                    if fi == 0:
                        acc_v[pl.ds(ao, 16)] = d * d
                    else:
                        acc_v[pl.ds(ao, 16)] = acc_v[pl.ds(ao, 16)] + d * d
                return 0

            lax.fori_loop(0, _BLK // (16 * _U), grp, 0)

    pltpu.sync_copy(acc_v, d2_out.at[w])


_sc_call = functools.partial(
    pl.kernel,
    out_type=jax.ShapeDtypeStruct((_NW, _BATCH), jnp.float32),
    mesh=plsc.VectorSubcoreMesh(
        core_axis_name="c", subcore_axis_name="s", num_cores=_NC, num_subcores=_NS
    ),
    compiler_params=pltpu.CompilerParams(needs_layout_passes=False),
    scratch_types=[
        pltpu.VMEM((_CLS,), jnp.float32),
        pltpu.VMEM((_BLK,), jnp.int32),
        pltpu.VMEM((_BLK,), jnp.float32),
        pltpu.VMEM((_BATCH,), jnp.float32),
        pltpu.SemaphoreType.DMA,
    ],
)(_sc_body)


def _tc_body(d2_ref, out_ref):
    t = jnp.sum(d2_ref[...], axis=0)
    out_ref[...] = jnp.sum(jnp.sqrt(t)).reshape(1, 1)


_tc_call = pl.pallas_call(
    _tc_body, out_shape=jax.ShapeDtypeStruct((1, 1), jnp.float32)
)


def kernel(x, labels, centre):
    d2 = _sc_call(centre.T, x.T, labels.astype(jnp.int32))
    return _tc_call(d2)[0, 0]


# speedup vs baseline: 1.8600x; 1.1513x over previous
import functools

import jax
import jax.numpy as jnp
from jax import lax
from jax.experimental import pallas as pl
from jax.experimental.pallas import tpu as pltpu
from jax.experimental.pallas import tpu_sc as plsc

_NC = 2
_NS = 16
_NW = _NC * _NS
_FEAT = 64
_BATCH = 16384
_CLS = 100000
_BLK = 2048
_NBLK = _BATCH // _BLK
_U = 16


def _sc_body(ct, xt, labels, d2_out, row_v, lab_v, x_v, acc_v, sem_r, sem_l, sem_x, sem_o):
    c = lax.axis_index("c")
    s = lax.axis_index("s")
    w = c * _NS + s

    pltpu.async_copy(labels, lab_v, sem_l).wait()

    out_cp = [None, None]
    for fi in range(2):
        f = w * 2 + fi
        cp_row = pltpu.async_copy(ct.at[f], row_v, sem_r)
        cp_x = [None, None]
        cp_x[0] = pltpu.async_copy(xt.at[f, pl.ds(0, _BLK)], x_v.at[0], sem_x)
        cp_row.wait()
        for b in range(_NBLK):
            cur = b % 2
            nxt = 1 - cur
            if b + 1 < _NBLK:
                cp_x[nxt] = pltpu.async_copy(
                    xt.at[f, pl.ds((b + 1) * _BLK, _BLK)], x_v.at[nxt], sem_x
                )
            cp_x[cur].wait()
            if out_cp[cur] is not None:
                out_cp[cur].wait()

            def grp(i, _, b=b, cur=cur):
                base = i * (16 * _U)
                for u in range(_U):
                    o = base + u * 16
                    lv = lab_v[pl.ds(b * _BLK + o, 16)]
                    xv = x_v[cur, pl.ds(o, 16)]
                    cv = plsc.load_gather(row_v, [lv])
                    d = xv - cv
                    acc_v[cur, pl.ds(o, 16)] = d * d
                return 0

            lax.fori_loop(0, _BLK // (16 * _U), grp, 0)
            out_cp[cur] = pltpu.async_copy(
                acc_v.at[cur], d2_out.at[f, pl.ds(b * _BLK, _BLK)], sem_o
            )
    for cp in out_cp:
        if cp is not None:
            cp.wait()


_sc_call = functools.partial(
    pl.kernel,
    out_type=jax.ShapeDtypeStruct((_FEAT, _BATCH), jnp.float32),
    mesh=plsc.VectorSubcoreMesh(
        core_axis_name="c", subcore_axis_name="s", num_cores=_NC, num_subcores=_NS
    ),
    compiler_params=pltpu.CompilerParams(needs_layout_passes=False),
    scratch_types=[
        pltpu.VMEM((_CLS,), jnp.float32),
        pltpu.VMEM((_BATCH,), jnp.int32),
        pltpu.VMEM((2, _BLK), jnp.float32),
        pltpu.VMEM((2, _BLK), jnp.float32),
        pltpu.SemaphoreType.DMA,
        pltpu.SemaphoreType.DMA,
        pltpu.SemaphoreType.DMA,
        pltpu.SemaphoreType.DMA,
    ],
)(_sc_body)


def _tc_body(d2_ref, out_ref):
    t = jnp.sum(d2_ref[...], axis=0)
    out_ref[...] = jnp.sum(jnp.sqrt(t)).reshape(1, 1)


_tc_call = pl.pallas_call(
    _tc_body, out_shape=jax.ShapeDtypeStruct((1, 1), jnp.float32)
)


def kernel(x, labels, centre):
    d2 = _sc_call(centre.T, x.T, labels.astype(jnp.int32))
    return _tc_call(d2)[0, 0]
